# baseline (speedup 1.0000x reference)
import functools

import jax
import jax.numpy as jnp
from jax import lax
from jax.experimental import pallas as pl
from jax.experimental.pallas import tpu as pltpu
from jax.experimental.pallas import tpu_sc as plsc

NC = 2
NS = 16
LANES = 16
BLK = 128
TCH = 512


def _sc_attention(k3, v3, q3, rg_flat, scale_vec, *, nh, hd, n_vox, topk):
    heads_per_core = nh // NC
    workers_per_head = NS // heads_per_core
    rows_per_w = n_vox // workers_per_head
    n_blocks = rows_per_w // BLK
    n_tch = rows_per_w // TCH
    hd2 = 2 * hd

    mesh = plsc.VectorSubcoreMesh(
        core_axis_name="c", subcore_axis_name="s",
        num_cores=NC, num_subcores=NS)

    @functools.partial(
        pl.kernel,
        out_type=[
            jax.ShapeDtypeStruct((nh, hd, n_vox), jnp.float32),
            jax.ShapeDtypeStruct((nh, n_vox, topk), jnp.float32),
        ],
        mesh=mesh,
        compiler_params=pltpu.CompilerParams(
            needs_layout_passes=False, use_tc_tiling_on_sc=False),
        scratch_types=[
            pltpu.HBM((nh * n_vox, hd2), jnp.float32),
            pltpu.VMEM((hd, TCH), jnp.float32),
            pltpu.VMEM((hd, TCH), jnp.float32),
            pltpu.VMEM((TCH, hd2), jnp.float32),
            pltpu.VMEM((hd, BLK), jnp.float32),
            pltpu.VMEM((topk * BLK,), jnp.int32),
            pltpu.VMEM((topk * BLK, hd2), jnp.float32),
            pltpu.VMEM((hd, BLK), jnp.float32),
            pltpu.VMEM((BLK, topk), jnp.float32),
            pltpu.VMEM((LANES,), jnp.float32),
            pltpu.SemaphoreType.DMA,
        ],
    )
    def attend(k_hbm, v_hbm, q_hbm, rg_hbm, scale_hbm,
               out_hbm, attn_hbm,
               kvT, kin, vin, kvt, qv, idxv, kvg, outv, attnv, scalev, sem):
        cid = lax.axis_index("c")
        sid = lax.axis_index("s")
        h = cid * heads_per_core + sid // workers_per_head
        base_n = (sid % workers_per_head) * rows_per_w
        row0 = h * n_vox + base_n

        pltpu.sync_copy(scale_hbm, scalev)
        iota = lax.iota(jnp.int32, LANES)

        def pack_chunk(ic, carry):
            n0 = base_n + ic * TCH
            pltpu.sync_copy(k_hbm.at[h, :, pl.ds(n0, TCH)], kin)
            pltpu.sync_copy(v_hbm.at[h, :, pl.ds(n0, TCH)], vin)

            def tpose(g, carry2):
                ccol = iota + g * LANES
                for d0 in range(0, hd, LANES):
                    for i in range(LANES):
                        dvec = d0 + ((iota + i) & (LANES - 1))
                        plsc.store_scatter(
                            kvt, [ccol, dvec],
                            plsc.load_gather(kin, [dvec, ccol]))
                        plsc.store_scatter(
                            kvt, [ccol, dvec + hd],
                            plsc.load_gather(vin, [dvec, ccol]))
                return carry2

            lax.fori_loop(0, TCH // LANES, tpose, 0)
            pltpu.sync_copy(kvt, kvT.at[pl.ds(h * n_vox + n0, TCH)])
            return carry

        lax.fori_loop(0, n_tch, pack_chunk, 0)
        plsc.subcore_barrier()

        def block(i, carry):
            n0 = base_n + i * BLK
            pltpu.sync_copy(q_hbm.at[h, :, pl.ds(n0, BLK)], qv)
            pltpu.sync_copy(
                rg_hbm.at[pl.ds((h * n_vox + n0) * topk, BLK * topk)], idxv)

            def addoff(j, carry2):
                o = j * LANES
                idxv[pl.ds(o, LANES)] = idxv[pl.ds(o, LANES)] + h * n_vox
                return carry2

            lax.fori_loop(0, (BLK * topk) // LANES, addoff, 0)

            handles = []
            for p in range(topk):
                handles.append(pltpu.async_copy(
                    kvT.at[idxv.at[pl.ds(p * BLK, BLK)]],
                    kvg.at[pl.ds(p * BLK, BLK)], sem))
            for hnd in handles:
                hnd.wait()

            sc = scalev[...]

            def comp(j, carry2):
                c0 = j * LANES
                ccol = iota + c0
                rowsl = [ccol * topk + t for t in range(topk)]
                svecs = [None] * topk
                for i2 in range(hd):
                    dvec = (iota + i2) & (hd - 1)
                    qg = plsc.load_gather(qv, [dvec, ccol])
                    for t in range(topk):
                        kgv = plsc.load_gather(kvg, [rowsl[t], dvec])
                        svecs[t] = (qg * kgv if svecs[t] is None
                                    else svecs[t] + qg * kgv)
                svecs = [s * sc for s in svecs]
                m = jnp.maximum(jnp.maximum(svecs[0], svecs[1]),
                                jnp.maximum(svecs[2], svecs[3]))
                evecs = [jnp.exp(s - m) for s in svecs]
                denom = evecs[0] + evecs[1] + evecs[2] + evecs[3]
                pvecs = [e / denom for e in evecs]
                for t in range(topk):
                    plsc.store_scatter(
                        attnv, [ccol, jnp.full((LANES,), t, jnp.int32)],
                        pvecs[t])
                for i2 in range(hd):
                    dvec = (iota + i2) & (hd - 1)
                    acc = pvecs[0] * plsc.load_gather(
                        kvg, [rowsl[0], dvec + hd])
                    for t in range(1, topk):
                        acc = acc + pvecs[t] * plsc.load_gather(
                            kvg, [rowsl[t], dvec + hd])
                    plsc.store_scatter(outv, [dvec, ccol], acc)
                return carry2

            lax.fori_loop(0, BLK // LANES, comp, 0)

            pltpu.sync_copy(outv, out_hbm.at[h, :, pl.ds(n0, BLK)])
            pltpu.sync_copy(attnv, attn_hbm.at[h, pl.ds(n0, BLK), :])
            return carry

        lax.fori_loop(0, n_blocks, block, 0)

    return attend(k3, v3, q3, rg_flat, scale_vec)


def kernel(mask, query, key, value, scale, region_graph):
    del mask
    b, ch, hh, ww, dd = query.shape
    _, nh, n_vox, topk = region_graph.shape
    hd = ch // nh
    assert b == 1 and n_vox == hh * ww * dd
    assert hd % LANES == 0 and (hd & (hd - 1)) == 0

    q3 = query.reshape(nh, hd, n_vox)
    k3 = key.reshape(nh, hd, n_vox)
    v3 = value.reshape(nh, hd, n_vox)
    rg_flat = region_graph.astype(jnp.int32).reshape(nh * n_vox * topk)
    scale_vec = jnp.broadcast_to(scale.astype(jnp.float32), (LANES,))

    out3, attn3 = _sc_attention(k3, v3, q3, rg_flat, scale_vec,
                                nh=nh, hd=hd, n_vox=n_vox, topk=topk)
    out = out3.reshape(b, ch, hh, ww, dd)
    attn = attn3.reshape(b, nh, n_vox, 1, topk)
    return out, attn


# speedup vs baseline: 1.9077x; 1.9077x over previous
import functools

import jax
import jax.numpy as jnp
from jax import lax
from jax.experimental import pallas as pl
from jax.experimental.pallas import tpu as pltpu
from jax.experimental.pallas import tpu_sc as plsc

NC = 2
NS = 16
LANES = 16
BLK = 128
HB = 64
TCH = 64


def _sc_attention(qs, kt, vt, rg3, *, nh, hd, n_vox, topk):
    ch = nh * hd
    cg = ch // NC
    hpc = nh // NC
    vox_per_w = n_vox // NS
    n_blocks = vox_per_w // BLK
    n_tch = vox_per_w // TCH
    hd2 = 2 * hd
    tbl_rows = nh * n_vox // 2

    mesh = plsc.VectorSubcoreMesh(
        core_axis_name="c", subcore_axis_name="s",
        num_cores=NC, num_subcores=NS)

    @functools.partial(
        pl.kernel,
        out_type=[
            jax.ShapeDtypeStruct((n_vox, ch), jnp.float32),
            jax.ShapeDtypeStruct((nh * n_vox * topk // BLK, BLK),
                                 jnp.float32),
        ],
        mesh=mesh,
        compiler_params=pltpu.CompilerParams(
            needs_layout_passes=False, use_tc_tiling_on_sc=True),
        scratch_types=[
            pltpu.HBM((tbl_rows, 2 * hd2), jnp.float32),
            pltpu.VMEM((TCH, cg), jnp.float32),
            pltpu.VMEM((TCH, cg), jnp.float32),
            pltpu.VMEM((hpc * TCH // 2, 2 * hd2), jnp.float32),
            pltpu.VMEM((BLK, cg), jnp.float32),
            pltpu.VMEM((topk, BLK), jnp.int32),
            pltpu.VMEM((topk, BLK), jnp.int32),
            pltpu.VMEM((topk, BLK), jnp.int32),
            pltpu.VMEM((topk * HB, 2 * hd2), jnp.float32),
            pltpu.VMEM((BLK, cg), jnp.float32),
            pltpu.VMEM((topk, BLK), jnp.float32),
            pltpu.SemaphoreType.DMA,
        ],
    )
    def attend(q_hbm, k_hbm, v_hbm, rg_hbm,
               out_hbm, attn_hbm,
               tbl, kin, vin, kvt, qv, rgv, jv, pv, kvg, outv, attnv, sem):
        cid = lax.axis_index("c")
        sid = lax.axis_index("s")
        base_v = sid * vox_per_w
        c0ch = cid * cg

        iota = lax.iota(jnp.int32, LANES)

        def pack_chunk(ic, carry):
            n0 = base_v + ic * TCH
            pltpu.sync_copy(k_hbm.at[pl.ds(n0, TCH), pl.ds(c0ch, cg)], kin)
            pltpu.sync_copy(v_hbm.at[pl.ds(n0, TCH), pl.ds(c0ch, cg)], vin)

            def assemble(j, carry2):
                for hh in range(hpc):
                    r = hh * (TCH // 2) + j
                    for d16 in range(hd // LANES):
                        o = d16 * LANES
                        kvt[r, pl.ds(o, LANES)] = (
                            kin[2 * j, pl.ds(hh * hd + o, LANES)])
                        kvt[r, pl.ds(hd + o, LANES)] = (
                            vin[2 * j, pl.ds(hh * hd + o, LANES)])
                        kvt[r, pl.ds(hd2 + o, LANES)] = (
                            kin[2 * j + 1, pl.ds(hh * hd + o, LANES)])
                        kvt[r, pl.ds(hd2 + hd + o, LANES)] = (
                            vin[2 * j + 1, pl.ds(hh * hd + o, LANES)])
                return carry2

            lax.fori_loop(0, TCH // 2, assemble, 0)
            whandles = []
            for hh in range(hpc):
                row0 = ((cid * hpc + hh) * n_vox + n0) // 2
                whandles.append(pltpu.async_copy(
                    kvt.at[pl.ds(hh * (TCH // 2), TCH // 2), :],
                    tbl.at[pl.ds(row0, TCH // 2), :], sem))
            for hnd in whandles:
                hnd.wait()
            return carry

        lax.fori_loop(0, n_tch, pack_chunk, 0)
        plsc.subcore_barrier()

        def block(i, carry):
            n0 = base_v + i * BLK
            pltpu.sync_copy(q_hbm.at[pl.ds(n0, BLK), pl.ds(c0ch, cg)], qv)

            for hh in range(hpc):
                h = cid * hpc + hh
                rrow = (h * (n_vox // BLK) + n0 // BLK) * topk
                pltpu.sync_copy(rg_hbm.at[pl.ds(rrow, topk), :], rgv)

                def addoff(j, carry2, _hh=hh, _h=h):
                    t = j // (BLK // LANES)
                    o = (j % (BLK // LANES)) * LANES
                    r = rgv[t, pl.ds(o, LANES)] + _h * n_vox
                    jv[t, pl.ds(o, LANES)] = r >> 1
                    pv[t, pl.ds(o, LANES)] = (r & 1) * hd2
                    return carry2

                lax.fori_loop(0, topk * (BLK // LANES), addoff, 0)

                for half in range(BLK // HB):
                    handles = []
                    for t in range(topk):
                        handles.append(pltpu.async_copy(
                            tbl.at[jv.at[t, pl.ds(half * HB, HB)]],
                            kvg.at[pl.ds(t * HB, HB)], sem))
                    for hnd in handles:
                        hnd.wait()

                    def comp(j, carry2, _hh=hh, _half=half):
                        c0 = j * LANES
                        ccol = iota + c0 + _half * HB
                        rowsl = [iota + c0 + t * HB for t in range(topk)]
                        pcols = [plsc.load_gather(
                            pv, [jnp.full((LANES,), t, jnp.int32), ccol])
                            for t in range(topk)]
                        svecs = [None] * topk
                        for i2 in range(hd):
                            dvec = (iota + i2) & (hd - 1)
                            qg = plsc.load_gather(qv, [ccol, dvec + _hh * hd])
                            for t in range(topk):
                                kgv = plsc.load_gather(
                                    kvg, [rowsl[t], pcols[t] + dvec])
                                svecs[t] = (qg * kgv if svecs[t] is None
                                            else svecs[t] + qg * kgv)
                        m = jnp.maximum(jnp.maximum(svecs[0], svecs[1]),
                                        jnp.maximum(svecs[2], svecs[3]))
                        evecs = [jnp.exp(s - m) for s in svecs]
                        denom = evecs[0] + evecs[1] + evecs[2] + evecs[3]
                        pvecs = [e / denom for e in evecs]
                        for t in range(topk):
                            attnv[t, pl.ds(c0 + _half * HB, LANES)] = pvecs[t]
                        for i2 in range(hd):
                            dvec = (iota + i2) & (hd - 1)
                            acc = pvecs[0] * plsc.load_gather(
                                kvg, [rowsl[0], pcols[0] + hd + dvec])
                            for t in range(1, topk):
                                acc = acc + pvecs[t] * plsc.load_gather(
                                    kvg, [rowsl[t], pcols[t] + hd + dvec])
                            plsc.store_scatter(
                                outv, [ccol, dvec + _hh * hd], acc)
                        return carry2

                    lax.fori_loop(0, HB // LANES, comp, 0)

                pltpu.sync_copy(attnv, attn_hbm.at[pl.ds(rrow, topk), :])

            pltpu.sync_copy(outv, out_hbm.at[pl.ds(n0, BLK), pl.ds(c0ch, cg)])
            return carry

        lax.fori_loop(0, n_blocks, block, 0)

    return attend(qs, kt, vt, rg3)


def kernel(mask, query, key, value, scale, region_graph):
    del mask
    b, ch, hh, ww, dd = query.shape
    _, nh, n_vox, topk = region_graph.shape
    hd = ch // nh
    assert b == 1 and n_vox == hh * ww * dd
    assert hd % LANES == 0 and (hd & (hd - 1)) == 0

    qs = (query * scale)[0].transpose(1, 2, 3, 0).reshape(n_vox, ch)
    kt = key[0].transpose(1, 2, 3, 0).reshape(n_vox, ch)
    vt = value[0].transpose(1, 2, 3, 0).reshape(n_vox, ch)
    rg2 = (region_graph[0].transpose(0, 2, 1)
           .reshape(nh, topk, n_vox // BLK, BLK)
           .transpose(0, 2, 1, 3)
           .reshape(nh * n_vox * topk // BLK, BLK).astype(jnp.int32))
    rg2 = jnp.concatenate(
        [rg2, jnp.zeros((n_vox - rg2.shape[0], BLK), jnp.int32)], axis=0)

    out2, attn2 = _sc_attention(qs, kt, vt, rg2,
                                nh=nh, hd=hd, n_vox=n_vox, topk=topk)
    out = out2.reshape(1, hh, ww, dd, ch).transpose(0, 4, 1, 2, 3)
    attn = (attn2.reshape(nh, n_vox // BLK, topk, BLK)
            .transpose(0, 1, 3, 2)
            .reshape(b, nh, n_vox, 1, topk))
    return out, attn


# speedup vs baseline: 1.9144x; 1.0035x over previous
import functools

import jax
import jax.numpy as jnp
from jax import lax
from jax.experimental import pallas as pl
from jax.experimental.pallas import tpu as pltpu
from jax.experimental.pallas import tpu_sc as plsc

NC = 2
NS = 16
LANES = 16
BLK = 128
HB = 64
TCH = 32


def _sc_attention(qs, kt, vt, rg3, *, nh, hd, n_vox, topk):
    ch = nh * hd
    cg = ch // NC
    hpc = nh // NC
    vox_per_w = n_vox // NS
    n_blocks = vox_per_w // BLK
    n_tch = vox_per_w // TCH
    hd2 = 2 * hd
    tbl_rows = nh * n_vox // 2

    mesh = plsc.VectorSubcoreMesh(
        core_axis_name="c", subcore_axis_name="s",
        num_cores=NC, num_subcores=NS)

    @functools.partial(
        pl.kernel,
        out_type=[
            jax.ShapeDtypeStruct((n_vox, ch), jnp.float32),
            jax.ShapeDtypeStruct((nh * n_vox * topk // BLK, BLK),
                                 jnp.float32),
        ],
        mesh=mesh,
        compiler_params=pltpu.CompilerParams(
            needs_layout_passes=False, use_tc_tiling_on_sc=True),
        scratch_types=[
            pltpu.HBM((tbl_rows, 2 * hd2), jnp.float32),
            pltpu.VMEM((TCH, cg), jnp.float32),
            pltpu.VMEM((TCH, cg), jnp.float32),
            pltpu.VMEM((hpc * TCH // 2, 2 * hd2), jnp.float32),
            pltpu.VMEM((BLK, cg), jnp.float32),
            pltpu.VMEM((topk, BLK), jnp.int32),
            pltpu.VMEM((topk, BLK), jnp.int32),
            pltpu.VMEM((topk, BLK), jnp.int32),
            pltpu.VMEM((2 * topk * HB, 2 * hd2), jnp.float32),
            pltpu.SemaphoreType.DMA,
            pltpu.VMEM((BLK, cg), jnp.float32),
            pltpu.VMEM((topk, BLK), jnp.float32),
            pltpu.SemaphoreType.DMA,
        ],
    )
    def attend(q_hbm, k_hbm, v_hbm, rg_hbm,
               out_hbm, attn_hbm,
               tbl, kin, vin, kvt, qv, rgv, jv, pv, kvg, sem2, outv, attnv,
               sem):
        cid = lax.axis_index("c")
        sid = lax.axis_index("s")
        base_v = sid * vox_per_w
        c0ch = cid * cg

        iota = lax.iota(jnp.int32, LANES)

        def pack_chunk(ic, carry):
            n0 = base_v + ic * TCH
            pltpu.sync_copy(k_hbm.at[pl.ds(n0, TCH), pl.ds(c0ch, cg)], kin)
            pltpu.sync_copy(v_hbm.at[pl.ds(n0, TCH), pl.ds(c0ch, cg)], vin)

            def assemble(j, carry2):
                for hh in range(hpc):
                    r = hh * (TCH // 2) + j
                    for d16 in range(hd // LANES):
                        o = d16 * LANES
                        kvt[r, pl.ds(o, LANES)] = (
                            kin[2 * j, pl.ds(hh * hd + o, LANES)])
                        kvt[r, pl.ds(hd + o, LANES)] = (
                            vin[2 * j, pl.ds(hh * hd + o, LANES)])
                        kvt[r, pl.ds(hd2 + o, LANES)] = (
                            kin[2 * j + 1, pl.ds(hh * hd + o, LANES)])
                        kvt[r, pl.ds(hd2 + hd + o, LANES)] = (
                            vin[2 * j + 1, pl.ds(hh * hd + o, LANES)])
                return carry2

            lax.fori_loop(0, TCH // 2, assemble, 0)
            whandles = []
            for hh in range(hpc):
                row0 = ((cid * hpc + hh) * n_vox + n0) // 2
                whandles.append(pltpu.async_copy(
                    kvt.at[pl.ds(hh * (TCH // 2), TCH // 2), :],
                    tbl.at[pl.ds(row0, TCH // 2), :], sem))
            for hnd in whandles:
                hnd.wait()
            return carry

        lax.fori_loop(0, n_tch, pack_chunk, 0)
        plsc.subcore_barrier()

        def block(i, carry):
            n0 = base_v + i * BLK
            pltpu.sync_copy(q_hbm.at[pl.ds(n0, BLK), pl.ds(c0ch, cg)], qv)

            for hh in range(hpc):
                h = cid * hpc + hh
                rrow = (h * (n_vox // BLK) + n0 // BLK) * topk
                pltpu.sync_copy(rg_hbm.at[pl.ds(rrow, topk), :], rgv)

                def addoff(j, carry2, _hh=hh, _h=h):
                    t = j // (BLK // LANES)
                    o = (j % (BLK // LANES)) * LANES
                    r = rgv[t, pl.ds(o, LANES)] + _h * n_vox
                    jv[t, pl.ds(o, LANES)] = r >> 1
                    pv[t, pl.ds(o, LANES)] = (r & 1) * hd2
                    return carry2

                lax.fori_loop(0, topk * (BLK // LANES), addoff, 0)

                hpair = []
                for half in range(BLK // HB):
                    hsem = sem if half == 0 else sem2
                    hnds = []
                    for t in range(topk):
                        hnds.append(pltpu.async_copy(
                            tbl.at[jv.at[t, pl.ds(half * HB, HB)]],
                            kvg.at[pl.ds((half * topk + t) * HB, HB)], hsem))
                    hpair.append(hnds)

                for half in range(BLK // HB):
                    for hnd in hpair[half]:
                        hnd.wait()

                    def comp(j, carry2, _hh=hh, _half=half):
                        c0 = j * LANES
                        ccol = iota + c0 + _half * HB
                        rowsl = [iota + c0 + (_half * topk + t) * HB
                                 for t in range(topk)]
                        pcols = [plsc.load_gather(
                            pv, [jnp.full((LANES,), t, jnp.int32), ccol])
                            for t in range(topk)]
                        svecs = [None] * topk
                        for i2 in range(hd):
                            dvec = (iota + i2) & (hd - 1)
                            qg = plsc.load_gather(qv, [ccol, dvec + _hh * hd])
                            for t in range(topk):
                                kgv = plsc.load_gather(
                                    kvg, [rowsl[t], pcols[t] + dvec])
                                svecs[t] = (qg * kgv if svecs[t] is None
                                            else svecs[t] + qg * kgv)
                        m = jnp.maximum(jnp.maximum(svecs[0], svecs[1]),
                                        jnp.maximum(svecs[2], svecs[3]))
                        evecs = [jnp.exp(s - m) for s in svecs]
                        denom = evecs[0] + evecs[1] + evecs[2] + evecs[3]
                        pvecs = [e / denom for e in evecs]
                        for t in range(topk):
                            attnv[t, pl.ds(c0 + _half * HB, LANES)] = pvecs[t]
                        for i2 in range(hd):
                            dvec = (iota + i2) & (hd - 1)
                            acc = pvecs[0] * plsc.load_gather(
                                kvg, [rowsl[0], pcols[0] + hd + dvec])
                            for t in range(1, topk):
                                acc = acc + pvecs[t] * plsc.load_gather(
                                    kvg, [rowsl[t], pcols[t] + hd + dvec])
                            plsc.store_scatter(
                                outv, [ccol, dvec + _hh * hd], acc)
                        return carry2

                    lax.fori_loop(0, HB // LANES, comp, 0)

                pltpu.sync_copy(attnv, attn_hbm.at[pl.ds(rrow, topk), :])

            pltpu.sync_copy(outv, out_hbm.at[pl.ds(n0, BLK), pl.ds(c0ch, cg)])
            return carry

        lax.fori_loop(0, n_blocks, block, 0)

    return attend(qs, kt, vt, rg3)


def kernel(mask, query, key, value, scale, region_graph):
    del mask
    b, ch, hh, ww, dd = query.shape
    _, nh, n_vox, topk = region_graph.shape
    hd = ch // nh
    assert b == 1 and n_vox == hh * ww * dd
    assert hd % LANES == 0 and (hd & (hd - 1)) == 0

    qs = (query * scale)[0].transpose(1, 2, 3, 0).reshape(n_vox, ch)
    kt = key[0].transpose(1, 2, 3, 0).reshape(n_vox, ch)
    vt = value[0].transpose(1, 2, 3, 0).reshape(n_vox, ch)
    rg2 = (region_graph[0].transpose(0, 2, 1)
           .reshape(nh, topk, n_vox // BLK, BLK)
           .transpose(0, 2, 1, 3)
           .reshape(nh * n_vox * topk // BLK, BLK).astype(jnp.int32))
    rg2 = jnp.concatenate(
        [rg2, jnp.zeros((n_vox - rg2.shape[0], BLK), jnp.int32)], axis=0)

    out2, attn2 = _sc_attention(qs, kt, vt, rg2,
                                nh=nh, hd=hd, n_vox=n_vox, topk=topk)
    out = out2.reshape(1, hh, ww, dd, ch).transpose(0, 4, 1, 2, 3)
    attn = (attn2.reshape(nh, n_vox // BLK, topk, BLK)
            .transpose(0, 1, 3, 2)
            .reshape(b, nh, n_vox, 1, topk))
    return out, attn


# speedup vs baseline: 2.1446x; 1.1202x over previous
import functools

import jax
import jax.numpy as jnp
from jax import lax
from jax.experimental import pallas as pl
from jax.experimental.pallas import tpu as pltpu
from jax.experimental.pallas import tpu_sc as plsc

NC = 2
NS = 16
LANES = 16
BLK = 128
HB = 64
TCH = 32


def _sc_attention(qs, kt, vt, rg3, *, nh, hd, n_vox, topk):
    ch = nh * hd
    cg = ch // NC
    hpc = nh // NC
    vox_per_w = n_vox // NS
    n_blocks = vox_per_w // BLK
    n_tch = vox_per_w // TCH
    hd2 = 2 * hd
    tbl_rows = nh * n_vox // 2

    mesh = plsc.VectorSubcoreMesh(
        core_axis_name="c", subcore_axis_name="s",
        num_cores=NC, num_subcores=NS)

    @functools.partial(
        pl.kernel,
        out_type=[
            jax.ShapeDtypeStruct((n_vox, ch), jnp.float32),
            jax.ShapeDtypeStruct((nh * n_vox * topk // BLK, BLK),
                                 jnp.float32),
        ],
        mesh=mesh,
        compiler_params=pltpu.CompilerParams(
            needs_layout_passes=False, use_tc_tiling_on_sc=True),
        scratch_types=[
            pltpu.HBM((tbl_rows, 2 * hd2), jnp.float32),
            pltpu.VMEM((TCH, cg), jnp.float32),
            pltpu.VMEM((TCH, cg), jnp.float32),
            pltpu.VMEM((TCH, cg), jnp.float32),
            pltpu.VMEM((TCH, cg), jnp.float32),
            pltpu.VMEM((hpc * TCH // 2, 2 * hd2), jnp.float32),
            pltpu.SemaphoreType.DMA,
            pltpu.SemaphoreType.DMA,
            pltpu.VMEM((BLK, cg), jnp.float32),
            pltpu.VMEM((topk, BLK), jnp.int32),
            pltpu.VMEM((topk, BLK), jnp.int32),
            pltpu.VMEM((topk, BLK), jnp.int32),
            pltpu.VMEM((2 * topk * HB, 2 * hd2), jnp.float32),
            pltpu.SemaphoreType.DMA,
            pltpu.VMEM((BLK, cg), jnp.float32),
            pltpu.VMEM((topk, BLK), jnp.float32),
            pltpu.SemaphoreType.DMA,
        ],
    )
    def attend(q_hbm, k_hbm, v_hbm, rg_hbm,
               out_hbm, attn_hbm,
               tbl, kina, vina, kinb, vinb, kvt, semia, semib,
               qv, rgv, jv, pv, kvg, sem2, outv, attnv, sem):
        cid = lax.axis_index("c")
        sid = lax.axis_index("s")
        base_v = sid * vox_per_w
        c0ch = cid * cg

        iota = lax.iota(jnp.int32, LANES)

        def issue_in(ic, dk, dv, s):
            n0 = base_v + ic * TCH
            pltpu.async_copy(k_hbm.at[pl.ds(n0, TCH), pl.ds(c0ch, cg)], dk, s)
            pltpu.async_copy(v_hbm.at[pl.ds(n0, TCH), pl.ds(c0ch, cg)], dv, s)

        def drain_in(dk, dv, s):
            pltpu.make_async_copy(
                k_hbm.at[pl.ds(0, TCH), pl.ds(0, cg)], dk, s).wait()
            pltpu.make_async_copy(
                k_hbm.at[pl.ds(0, TCH), pl.ds(0, cg)], dv, s).wait()

        def mk_assemble(kin, vin):
            def assemble(j, carry2):
                for hh in range(hpc):
                    r = hh * (TCH // 2) + j
                    for d16 in range(hd // LANES):
                        o = d16 * LANES
                        kvt[r, pl.ds(o, LANES)] = (
                            kin[2 * j, pl.ds(hh * hd + o, LANES)])
                        kvt[r, pl.ds(hd + o, LANES)] = (
                            vin[2 * j, pl.ds(hh * hd + o, LANES)])
                        kvt[r, pl.ds(hd2 + o, LANES)] = (
                            kin[2 * j + 1, pl.ds(hh * hd + o, LANES)])
                        kvt[r, pl.ds(hd2 + hd + o, LANES)] = (
                            vin[2 * j + 1, pl.ds(hh * hd + o, LANES)])
                return carry2
            return assemble

        def issue_writes(ic):
            n0 = base_v + ic * TCH
            handles = []
            for hh in range(hpc):
                row0 = ((cid * hpc + hh) * n_vox + n0) // 2
                handles.append(pltpu.async_copy(
                    kvt.at[pl.ds(hh * (TCH // 2), TCH // 2), :],
                    tbl.at[pl.ds(row0, TCH // 2), :], sem))
            return handles

        issue_in(0, kina, vina, semia)

        def pack_pair(k2, carry):
            ic0 = 2 * k2
            issue_in(ic0 + 1, kinb, vinb, semib)
            drain_in(kina, vina, semia)
            lax.fori_loop(0, TCH // 2, mk_assemble(kina, vina), 0)
            wa = issue_writes(ic0)
            issue_in(jnp.minimum(ic0 + 2, n_tch - 2), kina, vina, semia)
            drain_in(kinb, vinb, semib)
            for hnd in wa:
                hnd.wait()
            lax.fori_loop(0, TCH // 2, mk_assemble(kinb, vinb), 0)
            for hnd in issue_writes(ic0 + 1):
                hnd.wait()
            return carry

        lax.fori_loop(0, n_tch // 2, pack_pair, 0)
        drain_in(kina, vina, semia)
        plsc.subcore_barrier()

        def block(i, carry):
            n0 = base_v + i * BLK
            pltpu.sync_copy(q_hbm.at[pl.ds(n0, BLK), pl.ds(c0ch, cg)], qv)

            for hh in range(hpc):
                h = cid * hpc + hh
                rrow = (h * (n_vox // BLK) + n0 // BLK) * topk
                pltpu.sync_copy(rg_hbm.at[pl.ds(rrow, topk), :], rgv)

                def addoff(j, carry2, _hh=hh, _h=h):
                    t = j // (BLK // LANES)
                    o = (j % (BLK // LANES)) * LANES
                    r = rgv[t, pl.ds(o, LANES)] + _h * n_vox
                    jv[t, pl.ds(o, LANES)] = r >> 1
                    pv[t, pl.ds(o, LANES)] = (r & 1) * hd2
                    return carry2

                lax.fori_loop(0, topk * (BLK // LANES), addoff, 0)

                hpair = []
                for half in range(BLK // HB):
                    hsem = sem if half == 0 else sem2
                    hnds = []
                    for t in range(topk):
                        hnds.append(pltpu.async_copy(
                            tbl.at[jv.at[t, pl.ds(half * HB, HB)]],
                            kvg.at[pl.ds((half * topk + t) * HB, HB)], hsem))
                    hpair.append(hnds)

                for half in range(BLK // HB):
                    for hnd in hpair[half]:
                        hnd.wait()

                    def comp(j, carry2, _hh=hh, _half=half):
                        c0 = j * LANES
                        ccol = iota + c0 + _half * HB
                        rowsl = [iota + c0 + (_half * topk + t) * HB
                                 for t in range(topk)]
                        pcols = [plsc.load_gather(
                            pv, [jnp.full((LANES,), t, jnp.int32), ccol])
                            for t in range(topk)]
                        svecs = [None] * topk
                        for i2 in range(hd):
                            dvec = (iota + i2) & (hd - 1)
                            qg = plsc.load_gather(qv, [ccol, dvec + _hh * hd])
                            for t in range(topk):
                                kgv = plsc.load_gather(
                                    kvg, [rowsl[t], pcols[t] + dvec])
                                svecs[t] = (qg * kgv if svecs[t] is None
                                            else svecs[t] + qg * kgv)
                        m = jnp.maximum(jnp.maximum(svecs[0], svecs[1]),
                                        jnp.maximum(svecs[2], svecs[3]))
                        evecs = [jnp.exp(s - m) for s in svecs]
                        denom = evecs[0] + evecs[1] + evecs[2] + evecs[3]
                        pvecs = [e / denom for e in evecs]
                        for t in range(topk):
                            attnv[t, pl.ds(c0 + _half * HB, LANES)] = pvecs[t]
                        for i2 in range(hd):
                            dvec = (iota + i2) & (hd - 1)
                            acc = pvecs[0] * plsc.load_gather(
                                kvg, [rowsl[0], pcols[0] + hd + dvec])
                            for t in range(1, topk):
                                acc = acc + pvecs[t] * plsc.load_gather(
                                    kvg, [rowsl[t], pcols[t] + hd + dvec])
                            plsc.store_scatter(
                                outv, [ccol, dvec + _hh * hd], acc)
                        return carry2

                    lax.fori_loop(0, HB // LANES, comp, 0)

                pltpu.sync_copy(attnv, attn_hbm.at[pl.ds(rrow, topk), :])

            pltpu.sync_copy(outv, out_hbm.at[pl.ds(n0, BLK), pl.ds(c0ch, cg)])
            return carry

        lax.fori_loop(0, n_blocks, block, 0)

    return attend(qs, kt, vt, rg3)


def kernel(mask, query, key, value, scale, region_graph):
    del mask
    b, ch, hh, ww, dd = query.shape
    _, nh, n_vox, topk = region_graph.shape
    hd = ch // nh
    assert b == 1 and n_vox == hh * ww * dd
    assert hd % LANES == 0 and (hd & (hd - 1)) == 0

    qs = (query * scale)[0].transpose(1, 2, 3, 0).reshape(n_vox, ch)
    kt = key[0].transpose(1, 2, 3, 0).reshape(n_vox, ch)
    vt = value[0].transpose(1, 2, 3, 0).reshape(n_vox, ch)
    rg2 = (region_graph[0].transpose(0, 2, 1)
           .reshape(nh, topk, n_vox // BLK, BLK)
           .transpose(0, 2, 1, 3)
           .reshape(nh * n_vox * topk // BLK, BLK).astype(jnp.int32))
    rg2 = jnp.concatenate(
        [rg2, jnp.zeros((n_vox - rg2.shape[0], BLK), jnp.int32)], axis=0)

    out2, attn2 = _sc_attention(qs, kt, vt, rg2,
                                nh=nh, hd=hd, n_vox=n_vox, topk=topk)
    out = out2.reshape(1, hh, ww, dd, ch).transpose(0, 4, 1, 2, 3)
    attn = (attn2.reshape(nh, n_vox // BLK, topk, BLK)
            .transpose(0, 1, 3, 2)
            .reshape(b, nh, n_vox, 1, topk))
    return out, attn


# speedup vs baseline: 2.1479x; 1.0015x over previous
import functools

import jax
import jax.numpy as jnp
from jax import lax
from jax.experimental import pallas as pl
from jax.experimental.pallas import tpu as pltpu
from jax.experimental.pallas import tpu_sc as plsc

NC = 2
NS = 16
LANES = 16
BLK = 128
HB = 64
TCH = 32


def _sc_attention(qs, kt, vt, rg3, *, nh, hd, n_vox, topk):
    ch = nh * hd
    cg = ch // NC
    hpc = nh // NC
    vox_per_w = n_vox // NS
    n_blocks = vox_per_w // BLK
    n_tch = vox_per_w // TCH
    hd2 = 2 * hd
    tbl_rows = nh * n_vox // 2

    mesh = plsc.VectorSubcoreMesh(
        core_axis_name="c", subcore_axis_name="s",
        num_cores=NC, num_subcores=NS)

    @functools.partial(
        pl.kernel,
        out_type=[
            jax.ShapeDtypeStruct((n_vox, ch), jnp.float32),
            jax.ShapeDtypeStruct((nh * n_vox * topk // BLK, BLK),
                                 jnp.float32),
        ],
        mesh=mesh,
        compiler_params=pltpu.CompilerParams(
            needs_layout_passes=False, use_tc_tiling_on_sc=True),
        scratch_types=[
            pltpu.HBM((tbl_rows, 2 * hd2), jnp.float32),
            pltpu.VMEM((TCH, cg), jnp.float32),
            pltpu.VMEM((TCH, cg), jnp.float32),
            pltpu.VMEM((TCH, cg), jnp.float32),
            pltpu.VMEM((TCH, cg), jnp.float32),
            pltpu.VMEM((hpc * TCH // 2, 2 * hd2), jnp.float32),
            pltpu.SemaphoreType.DMA,
            pltpu.SemaphoreType.DMA,
            pltpu.VMEM((BLK, cg), jnp.float32),
            pltpu.VMEM((hpc * topk, BLK), jnp.int32),
            pltpu.VMEM((topk, BLK), jnp.int32),
            pltpu.VMEM((topk, BLK), jnp.int32),
            pltpu.VMEM((topk * HB, 2 * hd2), jnp.float32),
            pltpu.SemaphoreType.DMA,
            pltpu.VMEM((BLK, cg), jnp.float32),
            pltpu.VMEM((hpc * topk, BLK), jnp.float32),
            pltpu.SemaphoreType.DMA,
            pltpu.SemaphoreType.DMA,
        ],
    )
    def attend(q_hbm, k_hbm, v_hbm, rg_hbm,
               out_hbm, attn_hbm,
               tbl, kina, vina, kinb, vinb, kvt, semia, semib,
               qv, rgv, jv, pv, kvg, sem2, outv, attnv, semr, sem):
        cid = lax.axis_index("c")
        sid = lax.axis_index("s")
        base_v = sid * vox_per_w
        c0ch = cid * cg

        iota = lax.iota(jnp.int32, LANES)

        def issue_in(ic, dk, dv, s):
            n0 = base_v + ic * TCH
            pltpu.async_copy(k_hbm.at[pl.ds(n0, TCH), pl.ds(c0ch, cg)], dk, s)
            pltpu.async_copy(v_hbm.at[pl.ds(n0, TCH), pl.ds(c0ch, cg)], dv, s)

        def drain_in(dk, dv, s):
            pltpu.make_async_copy(
                k_hbm.at[pl.ds(0, TCH), pl.ds(0, cg)], dk, s).wait()
            pltpu.make_async_copy(
                k_hbm.at[pl.ds(0, TCH), pl.ds(0, cg)], dv, s).wait()

        def mk_assemble(kin, vin):
            def assemble(j, carry2):
                for hh in range(hpc):
                    r = hh * (TCH // 2) + j
                    for d16 in range(hd // LANES):
                        o = d16 * LANES
                        kvt[r, pl.ds(o, LANES)] = (
                            kin[2 * j, pl.ds(hh * hd + o, LANES)])
                        kvt[r, pl.ds(hd + o, LANES)] = (
                            vin[2 * j, pl.ds(hh * hd + o, LANES)])
                        kvt[r, pl.ds(hd2 + o, LANES)] = (
                            kin[2 * j + 1, pl.ds(hh * hd + o, LANES)])
                        kvt[r, pl.ds(hd2 + hd + o, LANES)] = (
                            vin[2 * j + 1, pl.ds(hh * hd + o, LANES)])
                return carry2
            return assemble

        def issue_writes(ic):
            n0 = base_v + ic * TCH
            handles = []
            for hh in range(hpc):
                row0 = ((cid * hpc + hh) * n_vox + n0) // 2
                handles.append(pltpu.async_copy(
                    kvt.at[pl.ds(hh * (TCH // 2), TCH // 2), :],
                    tbl.at[pl.ds(row0, TCH // 2), :], sem))
            return handles

        issue_in(0, kina, vina, semia)

        def pack_pair(k2, carry):
            ic0 = 2 * k2
            issue_in(ic0 + 1, kinb, vinb, semib)
            drain_in(kina, vina, semia)
            lax.fori_loop(0, TCH // 2, mk_assemble(kina, vina), 0)
            wa = issue_writes(ic0)
            issue_in(jnp.minimum(ic0 + 2, n_tch - 2), kina, vina, semia)
            drain_in(kinb, vinb, semib)
            for hnd in wa:
                hnd.wait()
            lax.fori_loop(0, TCH // 2, mk_assemble(kinb, vinb), 0)
            for hnd in issue_writes(ic0 + 1):
                hnd.wait()
            return carry

        lax.fori_loop(0, n_tch // 2, pack_pair, 0)
        drain_in(kina, vina, semia)
        plsc.subcore_barrier()

        def block(i, carry):
            n0 = base_v + i * BLK
            rhnds = []
            for hh in range(hpc):
                h = cid * hpc + hh
                rrow = (h * (n_vox // BLK) + n0 // BLK) * topk
                rhnds.append(pltpu.async_copy(
                    rg_hbm.at[pl.ds(rrow, topk), :],
                    rgv.at[pl.ds(hh * topk, topk), :], semr))
            pltpu.sync_copy(q_hbm.at[pl.ds(n0, BLK), pl.ds(c0ch, cg)], qv)
            for hnd in rhnds:
                hnd.wait()

            ahnds = []
            for hh in range(hpc):
                h = cid * hpc + hh
                rrow = (h * (n_vox // BLK) + n0 // BLK) * topk

                def addoff(j, carry2, _hh=hh, _h=h):
                    t = j // (BLK // LANES)
                    o = (j % (BLK // LANES)) * LANES
                    r = rgv[_hh * topk + t, pl.ds(o, LANES)] + _h * n_vox
                    jv[t, pl.ds(o, LANES)] = r >> 1
                    pv[t, pl.ds(o, LANES)] = (r & 1) * hd2
                    return carry2

                lax.fori_loop(0, topk * (BLK // LANES), addoff, 0)

                for half in range(BLK // HB):
                    hnds = []
                    for t in range(topk):
                        hnds.append(pltpu.async_copy(
                            tbl.at[jv.at[t, pl.ds(half * HB, HB)]],
                            kvg.at[pl.ds(t * HB, HB)], sem2))
                    for hnd in hnds:
                        hnd.wait()

                    def comp(j, carry2, _hh=hh, _half=half):
                        c0 = j * LANES
                        ccol = iota + c0 + _half * HB
                        rowsl = [iota + c0 + t * HB for t in range(topk)]
                        pcols = [plsc.load_gather(
                            pv, [jnp.full((LANES,), t, jnp.int32), ccol])
                            for t in range(topk)]
                        svecs = [None] * topk
                        for i2 in range(hd):
                            dvec = (iota + i2) & (hd - 1)
                            qg = plsc.load_gather(qv, [ccol, dvec + _hh * hd])
                            for t in range(topk):
                                kgv = plsc.load_gather(
                                    kvg, [rowsl[t], pcols[t] + dvec])
                                svecs[t] = (qg * kgv if svecs[t] is None
                                            else svecs[t] + qg * kgv)
                        m = jnp.maximum(jnp.maximum(svecs[0], svecs[1]),
                                        jnp.maximum(svecs[2], svecs[3]))
                        evecs = [jnp.exp(s - m) for s in svecs]
                        denom = evecs[0] + evecs[1] + evecs[2] + evecs[3]
                        pvecs = [e / denom for e in evecs]
                        for t in range(topk):
                            attnv[_hh * topk + t,
                                  pl.ds(c0 + _half * HB, LANES)] = pvecs[t]
                        for i2 in range(hd):
                            dvec = (iota + i2) & (hd - 1)
                            acc = pvecs[0] * plsc.load_gather(
                                kvg, [rowsl[0], pcols[0] + hd + dvec])
                            for t in range(1, topk):
                                acc = acc + pvecs[t] * plsc.load_gather(
                                    kvg, [rowsl[t], pcols[t] + hd + dvec])
                            plsc.store_scatter(
                                outv, [ccol, dvec + _hh * hd], acc)
                        return carry2

                    lax.fori_loop(0, HB // LANES, comp, 0)

                ahnds.append(pltpu.async_copy(
                    attnv.at[pl.ds(hh * topk, topk), :],
                    attn_hbm.at[pl.ds(rrow, topk), :], semr))

            pltpu.sync_copy(outv, out_hbm.at[pl.ds(n0, BLK), pl.ds(c0ch, cg)])
            for hnd in ahnds:
                hnd.wait()
            return carry

        lax.fori_loop(0, n_blocks, block, 0)

    return attend(qs, kt, vt, rg3)


def kernel(mask, query, key, value, scale, region_graph):
    del mask
    b, ch, hh, ww, dd = query.shape
    _, nh, n_vox, topk = region_graph.shape
    hd = ch // nh
    assert b == 1 and n_vox == hh * ww * dd
    assert hd % LANES == 0 and (hd & (hd - 1)) == 0

    qs = (query * scale)[0].transpose(1, 2, 3, 0).reshape(n_vox, ch)
    kt = key[0].transpose(1, 2, 3, 0).reshape(n_vox, ch)
    vt = value[0].transpose(1, 2, 3, 0).reshape(n_vox, ch)
    rg2 = (region_graph[0].transpose(0, 2, 1)
           .reshape(nh, topk, n_vox // BLK, BLK)
           .transpose(0, 2, 1, 3)
           .reshape(nh * n_vox * topk // BLK, BLK).astype(jnp.int32))
    rg2 = jnp.concatenate(
        [rg2, jnp.zeros((n_vox - rg2.shape[0], BLK), jnp.int32)], axis=0)

    out2, attn2 = _sc_attention(qs, kt, vt, rg2,
                                nh=nh, hd=hd, n_vox=n_vox, topk=topk)
    out = out2.reshape(1, hh, ww, dd, ch).transpose(0, 4, 1, 2, 3)
    attn = (attn2.reshape(nh, n_vox // BLK, topk, BLK)
            .transpose(0, 1, 3, 2)
            .reshape(b, nh, n_vox, 1, topk))
    return out, attn
